# baseline (speedup 1.0000x reference)
import functools

import jax
import jax.numpy as jnp
from jax.experimental import pallas as pl
from jax.experimental.pallas import tpu as pltpu

B, N, D, K = 8, 2048, 32, 128
ROW = 256
BIG = 1e6
LOG2E = 1.4426950408889634

_dot = functools.partial(
    jax.lax.dot_general, preferred_element_type=jnp.float32
)


def _oc_kernel(beta_ref, emb_ref, sid_ref, cp_ref, cpc_ref, out_ref, acc_ref):
    b = pl.program_id(0)

    @pl.when(b == 0)
    def _init():
        acc_ref[0] = 0.0
        acc_ref[1] = 0.0

    sid = sid_ref[0]
    cp = cp_ref[0] == 1
    x = beta_ref[0]
    E = emb_ref[0]
    cpc = (cpc_ref[0] == 1).astype(jnp.float32)

    cpf = cp.astype(jnp.float32)
    pos = jnp.sum(cpf)
    neg = jnp.float32(N) - pos

    pos_w = neg / (pos + 1e-6)
    w = jnp.where(cp, pos_w, 1.0)
    bce = jnp.maximum(x, 0.0) - x * cpf + jnp.log1p(jnp.exp(-jnp.abs(x)))
    beta_loss = jnp.sum(w * bce) * (1.0 / jnp.float32(N))

    kk = jax.lax.broadcasted_iota(jnp.int32, (K, N), 0)
    nn = jax.lax.broadcasted_iota(jnp.int32, (K, N), 1)
    M = sid == kk
    Mf = M.astype(jnp.float32)
    cpm = M & cp
    first = jnp.min(jnp.where(cpm, nn, N), axis=1, keepdims=True)
    has = (first < N).astype(jnp.float32)
    Ff = (nn == first).astype(jnp.float32)

    sq_col = jnp.sum(E * E, axis=1, keepdims=True)
    ones_col = jnp.ones((N, 1), jnp.float32)
    A = jnp.concatenate([E, sq_col, ones_col], axis=1)
    MF2 = jnp.concatenate([Mf, Ff], axis=0)
    SA2 = _dot(MF2, A, (((1,), (0,)), ((), ())))
    S1 = SA2[:K, :D]
    Ssq = SA2[:K, D:D + 1]
    cnt = SA2[:K, D + 1:D + 2]
    C = SA2[K:, :D]
    csq = SA2[K:, D:D + 1]
    cross = jnp.sum(C * S1, axis=1, keepdims=True)
    safe_cnt = jnp.maximum(cnt, 1.0)
    terms = has * (Ssq + cnt * csq - 2.0 * cross) / safe_cnt
    attraction = jnp.sum(terms)

    na = -(sq_col + BIG * (1.0 - cpc))
    X = jnp.concatenate([(2.0 * LOG2E) * E, LOG2E * na, ones_col], axis=1)
    Y = jnp.concatenate([E, ones_col, LOG2E * na], axis=1)

    rep_sum = jnp.float32(0.0)
    for i in range(N // ROW):
        r0 = i * ROW
        Xi = X[r0:r0 + ROW, :]
        Yi = Y[r0:, :]
        m = _dot(Xi, Yi, (((1,), (1,)), ((), ())))
        e = jnp.exp2(m)
        rep_sum = rep_sum + jnp.sum(e[:, :ROW])
        if r0 + ROW < N:
            rep_sum = rep_sum + 2.0 * jnp.sum(e[:, ROW:])
    repulsion = jnp.where(pos > 1.0, rep_sum / (pos * pos), 0.0)

    loss_b = beta_loss + attraction + repulsion
    valid = (pos >= 1.0) & (neg >= 1.0)
    acc_ref[0] += jnp.where(valid, loss_b, 0.0)
    acc_ref[1] += valid.astype(jnp.float32)

    @pl.when(b == B - 1)
    def _fin():
        cnt_v = acc_ref[1]
        out_ref[0, 0] = jnp.where(cnt_v == 0.0, 0.0,
                                  acc_ref[0] / jnp.maximum(cnt_v, 1.0))


@jax.jit
def kernel(beta, embed, slice_id, is_cp):
    beta2 = jnp.reshape(beta, (B, 1, N))
    sid2 = jnp.reshape(slice_id, (B, 1, N))
    cp2 = jnp.reshape(is_cp, (B, 1, N))
    cpc = jnp.reshape(is_cp, (B, N, 1))
    out = pl.pallas_call(
        _oc_kernel,
        grid=(B,),
        in_specs=[
            pl.BlockSpec((1, 1, N), lambda b: (b, 0, 0)),
            pl.BlockSpec((1, N, D), lambda b: (b, 0, 0)),
            pl.BlockSpec((1, 1, N), lambda b: (b, 0, 0)),
            pl.BlockSpec((1, 1, N), lambda b: (b, 0, 0)),
            pl.BlockSpec((1, N, 1), lambda b: (b, 0, 0)),
        ],
        out_specs=pl.BlockSpec(memory_space=pltpu.SMEM),
        out_shape=jax.ShapeDtypeStruct((1, 1), jnp.float32),
        scratch_shapes=[pltpu.SMEM((2,), jnp.float32)],
    )(beta2, embed, sid2, cp2, cpc)
    return out[0, 0]


# speedup vs baseline: 1.1121x; 1.1121x over previous
import functools

import jax
import jax.numpy as jnp
from jax.experimental import pallas as pl
from jax.experimental.pallas import tpu as pltpu

B, N, D, K = 8, 2048, 32, 128
ROW = 256
BIG = 1e6
LOG2E = 1.4426950408889634

_dot = functools.partial(
    jax.lax.dot_general, preferred_element_type=jnp.float32
)


def _oc_kernel(beta_ref, emb_ref, sid_ref, cp_ref, cpc_ref, out_ref, acc_ref):
    b = pl.program_id(0)

    @pl.when(b == 0)
    def _init():
        acc_ref[0] = 0.0
        acc_ref[1] = 0.0

    sid = sid_ref[0]
    cp = cp_ref[0] == 1
    x = beta_ref[0]
    E = emb_ref[0]
    cpc = (cpc_ref[0] == 1).astype(jnp.float32)

    cpf = cp.astype(jnp.float32)
    pos = jnp.sum(cpf)
    neg = jnp.float32(N) - pos

    pos_w = neg / (pos + 1e-6)
    w = jnp.where(cp, pos_w, 1.0)
    bce = jnp.maximum(x, 0.0) - x * cpf + jnp.log1p(jnp.exp(-jnp.abs(x)))
    beta_loss = jnp.sum(w * bce) * (1.0 / jnp.float32(N))

    kk = jax.lax.broadcasted_iota(jnp.int32, (K, N), 0)
    nn = jax.lax.broadcasted_iota(jnp.int32, (K, N), 1)
    M = sid == kk
    Mf = M.astype(jnp.float32)
    cpm = M & cp
    first = jnp.min(jnp.where(cpm, nn, N), axis=1, keepdims=True)
    has = (first < N).astype(jnp.float32)
    Ff = (nn == first).astype(jnp.float32)

    sq_col = jnp.sum(E * E, axis=1, keepdims=True)
    ones_col = jnp.ones((N, 1), jnp.float32)
    A = jnp.concatenate([E, sq_col, ones_col], axis=1)
    SA = _dot(Mf, A, (((1,), (0,)), ((), ())))
    CA = _dot(Ff, A, (((1,), (0,)), ((), ())))
    S1 = SA[:, :D]
    Ssq = SA[:, D:D + 1]
    cnt = SA[:, D + 1:D + 2]
    C = CA[:, :D]
    csq = CA[:, D:D + 1]
    cross = jnp.sum(C * S1, axis=1, keepdims=True)
    safe_cnt = jnp.maximum(cnt, 1.0)
    terms = has * (Ssq + cnt * csq - 2.0 * cross) / safe_cnt
    attraction = jnp.sum(terms)

    na = -(sq_col + BIG * (1.0 - cpc))
    X = jnp.concatenate([(2.0 * LOG2E) * E, LOG2E * na, ones_col], axis=1)
    Y = jnp.concatenate([E, ones_col, LOG2E * na], axis=1)

    rep_sum = jnp.float32(0.0)
    for i in range(N // ROW):
        r0 = i * ROW
        Xi = X[r0:r0 + ROW, :]
        Yi = Y[r0:, :]
        m = _dot(Xi, Yi, (((1,), (1,)), ((), ())))
        e = jnp.exp2(m)
        rep_sum = rep_sum + jnp.sum(e[:, :ROW])
        if r0 + ROW < N:
            rep_sum = rep_sum + 2.0 * jnp.sum(e[:, ROW:])
    repulsion = jnp.where(pos > 1.0, rep_sum / (pos * pos), 0.0)

    loss_b = beta_loss + attraction + repulsion
    valid = (pos >= 1.0) & (neg >= 1.0)
    acc_ref[0] += jnp.where(valid, loss_b, 0.0)
    acc_ref[1] += valid.astype(jnp.float32)

    @pl.when(b == B - 1)
    def _fin():
        cnt_v = acc_ref[1]
        out_ref[0, 0] = jnp.where(cnt_v == 0.0, 0.0,
                                  acc_ref[0] / jnp.maximum(cnt_v, 1.0))


@jax.jit
def kernel(beta, embed, slice_id, is_cp):
    beta2 = jnp.reshape(beta, (B, 1, N))
    sid2 = jnp.reshape(slice_id, (B, 1, N))
    cp2 = jnp.reshape(is_cp, (B, 1, N))
    cpc = jnp.reshape(is_cp, (B, N, 1))
    out = pl.pallas_call(
        _oc_kernel,
        grid=(B,),
        in_specs=[
            pl.BlockSpec((1, 1, N), lambda b: (b, 0, 0)),
            pl.BlockSpec((1, N, D), lambda b: (b, 0, 0)),
            pl.BlockSpec((1, 1, N), lambda b: (b, 0, 0)),
            pl.BlockSpec((1, 1, N), lambda b: (b, 0, 0)),
            pl.BlockSpec((1, N, 1), lambda b: (b, 0, 0)),
        ],
        out_specs=pl.BlockSpec(memory_space=pltpu.SMEM),
        out_shape=jax.ShapeDtypeStruct((1, 1), jnp.float32),
        scratch_shapes=[pltpu.SMEM((2,), jnp.float32)],
    )(beta2, embed, sid2, cp2, cpc)
    return out[0, 0]
